# baseline (speedup 1.0000x reference)
import functools
import jax
import jax.numpy as jnp
from jax import lax
from jax.experimental import pallas as pl
from jax.experimental.pallas import tpu as pltpu
from jax.experimental.pallas import tpu_sc as plsc

D_MODEL_ = 768
N_EXP_ = 64
NC_, NS_, L_ = 2, 16, 16
NW_ = NC_ * NS_


def _matmul_t_body(x_ref, wg_ref, lt_ref):
    bt = x_ref.shape[0]
    logits = lax.dot_general(
        x_ref[...], wg_ref[...],
        (((1,), (1,)), ((), ())),
        preferred_element_type=jnp.float32)
    lt = logits.T
    v = jnp.reshape(lt, (8, 8, bt // 128, 128))
    v = jnp.transpose(v, (0, 2, 1, 3))
    lt_ref[...] = v[None]


def _tc_logits_t(xf, wg, B, S, T):
    BT = 8192
    bpb = S // BT
    return pl.pallas_call(
        _matmul_t_body,
        grid=(T // BT,),
        in_specs=[
            pl.BlockSpec((BT, D_MODEL_), lambda i: (i, 0)),
            pl.BlockSpec((N_EXP_, D_MODEL_), lambda i: (0, 0)),
        ],
        out_specs=pl.BlockSpec(
            (1, 8, BT // 128, 8, 128),
            lambda i: (i // bpb, 0, i % bpb, 0, 0)),
        out_shape=jax.ShapeDtypeStruct((B, 8, S // 128, 8, 128), jnp.float32),
    )(xf, wg)


def _sc_topk(logits_flat, B, S, T):
    per_w = T // NW_
    n_groups = per_w // L_
    n_chains = 4
    chunks_per_w = per_w // 128

    mesh = plsc.VectorSubcoreMesh(core_axis_name="c", subcore_axis_name="s")

    @functools.partial(
        pl.kernel,
        mesh=mesh,
        out_type=[
            jax.ShapeDtypeStruct((2 * T,), jnp.float32),
            jax.ShapeDtypeStruct((2 * T,), jnp.int32),
        ],
        scratch_types=[
            pltpu.VMEM((per_w * N_EXP_,), jnp.float32),
            pltpu.VMEM((2 * per_w,), jnp.float32),
            pltpu.VMEM((2 * per_w,), jnp.int32),
            pltpu.SemaphoreType.DMA,
        ],
        compiler_params=pltpu.CompilerParams(needs_layout_passes=False),
    )
    def topk_kernel(logits_hbm, w_hbm, i_hbm, lv, wv, iv, sem):
        wid = lax.axis_index("s") * NC_ + lax.axis_index("c")
        base = wid * per_w
        b = base // S
        c0 = (base % S) // 128
        handles = []
        for j in range(8):
            off = b * (S * N_EXP_) + j * (S * 8) + c0 * 1024
            handles.append(pltpu.async_copy(
                logits_hbm.at[pl.ds(off, 8 * per_w)],
                lv.at[pl.ds(j * 8 * per_w, 8 * per_w)], sem))
        for h in handles:
            h.wait()

        neg_inf = jnp.full((L_,), -jnp.inf, jnp.float32)
        zeros_i = jnp.zeros((L_,), jnp.int32)

        def merge(m1a, i1a, m2a, i2a, m1b, i1b, m2b, i2b):
            gt = m1b > m1a
            M1 = jnp.where(gt, m1b, m1a)
            I1 = jnp.where(gt, i1b, i1a)
            Ml = jnp.where(gt, m1a, m1b)
            Il = jnp.where(gt, i1a, i1b)
            M2w = jnp.where(gt, m2b, m2a)
            I2w = jnp.where(gt, i2b, i2a)
            gt2 = Ml > M2w
            M2 = jnp.where(gt2, Ml, M2w)
            I2 = jnp.where(gt2, Il, I2w)
            return M1, I1, M2, I2

        def group_body(g, _):
            goff = (g // 8) * 1024 + (g % 8) * L_
            chains = []
            for c in range(n_chains):
                m1, m2 = neg_inf, neg_inf
                i1, i2 = zeros_i, zeros_i
                for jj in (2 * c, 2 * c + 1):
                    for r in range(8):
                        e = jj * 8 + r
                        v = lv[pl.ds(jj * 8192 + r * 128 + goff, L_)]
                        e_vec = jnp.full((L_,), e, jnp.int32)
                        gt1 = v > m1
                        gt2 = v > m2
                        m2n = jnp.where(gt1, m1, jnp.where(gt2, v, m2))
                        i2n = jnp.where(gt1, i1, jnp.where(gt2, e_vec, i2))
                        m1 = jnp.where(gt1, v, m1)
                        i1 = jnp.where(gt1, e_vec, i1)
                        m2, i2 = m2n, i2n
                chains.append((m1, i1, m2, i2))
            a = merge(*chains[0], *chains[1])
            bb = merge(*chains[2], *chains[3])
            m1, i1, m2, i2 = merge(*a, *bb)

            ex = jnp.exp(m2 - m1)
            denom = 1.0 + ex
            pos = (g // 8) * 256 + (g % 8) * L_
            wv[pl.ds(pos, L_)] = 1.0 / denom
            wv[pl.ds(pos + 128, L_)] = ex / denom
            iv[pl.ds(pos, L_)] = i1
            iv[pl.ds(pos + 128, L_)] = i2
            return ()

        lax.fori_loop(0, n_groups, group_body, ())

        pltpu.sync_copy(wv, w_hbm.at[pl.ds(2 * base, 2 * per_w)])
        pltpu.sync_copy(iv, i_hbm.at[pl.ds(2 * base, 2 * per_w)])

    return topk_kernel(logits_flat)


def kernel(x, W_gate):
    B, S, D = x.shape
    T = B * S
    xf = x.reshape(T, D)

    lt5 = _tc_logits_t(xf, W_gate, B, S, T)
    w_flat, i_flat = _sc_topk(lt5.reshape(T * N_EXP_), B, S, T)

    logits = jnp.transpose(lt5, (0, 2, 4, 1, 3)).reshape(B, S, N_EXP_)
    weights = jnp.transpose(
        w_flat.reshape(B, S // 128, 2, 128), (0, 1, 3, 2)).reshape(B, S, 2)
    indices = jnp.transpose(
        i_flat.reshape(B, S // 128, 2, 128), (0, 1, 3, 2)).reshape(B, S, 2)
    return (weights, indices, logits)


# speedup vs baseline: 1.0261x; 1.0261x over previous
import functools
import jax
import jax.numpy as jnp
from jax import lax
from jax.experimental import pallas as pl
from jax.experimental.pallas import tpu as pltpu
from jax.experimental.pallas import tpu_sc as plsc

D_MODEL_ = 768
N_EXP_ = 64
NC_, NS_, L_ = 2, 16, 16
NW_ = NC_ * NS_


def _matmul_t_body(x_ref, wg_ref, lt_ref):
    bt = x_ref.shape[0]
    logits = lax.dot_general(
        x_ref[...], wg_ref[...],
        (((1,), (1,)), ((), ())),
        preferred_element_type=jnp.float32)
    lt = logits.T
    v = jnp.reshape(lt, (8, 8, bt // 128, 128))
    v = jnp.transpose(v, (0, 2, 1, 3))
    lt_ref[...] = v[None]


def _tc_logits_t(xf, wg, B, S, T):
    BT = 4096
    bpb = S // BT
    return pl.pallas_call(
        _matmul_t_body,
        grid=(T // BT,),
        in_specs=[
            pl.BlockSpec((BT, D_MODEL_), lambda i: (i, 0)),
            pl.BlockSpec((N_EXP_, D_MODEL_), lambda i: (0, 0)),
        ],
        out_specs=pl.BlockSpec(
            (1, 8, BT // 128, 8, 128),
            lambda i: (i // bpb, 0, i % bpb, 0, 0)),
        out_shape=jax.ShapeDtypeStruct((B, 8, S // 128, 8, 128), jnp.float32),
    )(xf, wg)


def _sc_topk(logits_flat, B, S, T):
    per_w = T // NW_
    n_groups = per_w // L_
    n_chains = 4
    chunks_per_w = per_w // 128

    mesh = plsc.VectorSubcoreMesh(core_axis_name="c", subcore_axis_name="s")

    @functools.partial(
        pl.kernel,
        mesh=mesh,
        out_type=[
            jax.ShapeDtypeStruct((2 * T,), jnp.float32),
            jax.ShapeDtypeStruct((2 * T,), jnp.int32),
        ],
        scratch_types=[
            pltpu.VMEM((per_w * N_EXP_,), jnp.float32),
            pltpu.VMEM((2 * per_w,), jnp.float32),
            pltpu.VMEM((2 * per_w,), jnp.int32),
            pltpu.SemaphoreType.DMA,
        ],
        compiler_params=pltpu.CompilerParams(needs_layout_passes=False),
    )
    def topk_kernel(logits_hbm, w_hbm, i_hbm, lv, wv, iv, sem):
        wid = lax.axis_index("s") * NC_ + lax.axis_index("c")
        base = wid * per_w
        b = base // S
        c0 = (base % S) // 128
        handles = []
        for j in range(8):
            off = b * (S * N_EXP_) + j * (S * 8) + c0 * 1024
            handles.append(pltpu.async_copy(
                logits_hbm.at[pl.ds(off, 8 * per_w)],
                lv.at[pl.ds(j * 8 * per_w, 8 * per_w)], sem))
        for h in handles:
            h.wait()

        neg_inf = jnp.full((L_,), -jnp.inf, jnp.float32)
        zeros_i = jnp.zeros((L_,), jnp.int32)

        def merge(m1a, i1a, m2a, i2a, m1b, i1b, m2b, i2b):
            gt = m1b > m1a
            M1 = jnp.where(gt, m1b, m1a)
            I1 = jnp.where(gt, i1b, i1a)
            Ml = jnp.where(gt, m1a, m1b)
            Il = jnp.where(gt, i1a, i1b)
            M2w = jnp.where(gt, m2b, m2a)
            I2w = jnp.where(gt, i2b, i2a)
            gt2 = Ml > M2w
            M2 = jnp.where(gt2, Ml, M2w)
            I2 = jnp.where(gt2, Il, I2w)
            return M1, I1, M2, I2

        def group_body(g, _):
            goff = (g // 8) * 1024 + (g % 8) * L_
            chains = []
            for c in range(n_chains):
                m1, m2 = neg_inf, neg_inf
                i1, i2 = zeros_i, zeros_i
                for jj in (2 * c, 2 * c + 1):
                    for r in range(8):
                        e = jj * 8 + r
                        v = lv[pl.ds(jj * 8192 + r * 128 + goff, L_)]
                        e_vec = jnp.full((L_,), e, jnp.int32)
                        gt1 = v > m1
                        gt2 = v > m2
                        m2n = jnp.where(gt1, m1, jnp.where(gt2, v, m2))
                        i2n = jnp.where(gt1, i1, jnp.where(gt2, e_vec, i2))
                        m1 = jnp.where(gt1, v, m1)
                        i1 = jnp.where(gt1, e_vec, i1)
                        m2, i2 = m2n, i2n
                chains.append((m1, i1, m2, i2))
            a = merge(*chains[0], *chains[1])
            bb = merge(*chains[2], *chains[3])
            m1, i1, m2, i2 = merge(*a, *bb)

            ex = jnp.exp(m2 - m1)
            denom = 1.0 + ex
            pos = (g // 8) * 256 + (g % 8) * L_
            wv[pl.ds(pos, L_)] = 1.0 / denom
            wv[pl.ds(pos + 128, L_)] = ex / denom
            iv[pl.ds(pos, L_)] = i1
            iv[pl.ds(pos + 128, L_)] = i2
            return ()

        lax.fori_loop(0, n_groups, group_body, ())

        pltpu.sync_copy(wv, w_hbm.at[pl.ds(2 * base, 2 * per_w)])
        pltpu.sync_copy(iv, i_hbm.at[pl.ds(2 * base, 2 * per_w)])

    return topk_kernel(logits_flat)


def kernel(x, W_gate):
    B, S, D = x.shape
    T = B * S
    xf = x.reshape(T, D)

    lt5 = _tc_logits_t(xf, W_gate, B, S, T)
    w_flat, i_flat = _sc_topk(lt5.reshape(T * N_EXP_), B, S, T)

    logits = jnp.transpose(lt5, (0, 2, 4, 1, 3)).reshape(B, S, N_EXP_)
    weights = jnp.transpose(
        w_flat.reshape(B, S // 128, 2, 128), (0, 1, 3, 2)).reshape(B, S, 2)
    indices = jnp.transpose(
        i_flat.reshape(B, S // 128, 2, 128), (0, 1, 3, 2)).reshape(B, S, 2)
    return (weights, indices, logits)
